# baseline (speedup 1.0000x reference)
import functools

import jax
import jax.numpy as jnp
from jax import lax
from jax.experimental import pallas as pl
from jax.experimental.pallas import tpu as pltpu
from jax.experimental.pallas import tpu_sc as plsc

N = 10000
F = 1040
H = 8
C = 130
NT = 9
FPP = NT * 128
G = 64
ET = 170000
NW = 32
EPT = 6144
EPAD = EPT * NW
RPT = EPT // 128
NBK = 400
NGRID = N // NBK
NPT = N // 16
EPT2 = EPAD // 16
RPT2 = EPT2 // 128


def _k1_body(h_ref, w_ref, as_ref, ad_ref, xwt_ref, ats_ref, atd_ref,
             mx_ref):
    i = pl.program_id(0)
    xw = jnp.dot(h_ref[...], w_ref[...], preferred_element_type=jnp.float32,
        precision=lax.Precision.HIGHEST)
    a_s = jnp.dot(xw, as_ref[...], preferred_element_type=jnp.float32,
        precision=lax.Precision.HIGHEST)
    a_d = jnp.dot(xw, ad_ref[...], preferred_element_type=jnp.float32,
        precision=lax.Precision.HIGHEST)
    zpad = jnp.zeros((NBK, 112), jnp.float32)
    ats_ref[...] = jnp.concatenate([a_s, zpad], axis=1)
    atd_ref[...] = jnp.concatenate([a_d, zpad], axis=1)
    xwp = jnp.concatenate([xw, jnp.zeros((NBK, FPP - F), jnp.float32)],
                          axis=1)
    for j in range(NT):
        xwt_ref[j, :, :] = xwp[:, j * 128:(j + 1) * 128]

    @pl.when(i == 0)
    def _():
        mx_ref[...] = jnp.full((8, 128), -jnp.inf, jnp.float32)

    mx = mx_ref[...]
    z112 = jnp.zeros((1, 112), jnp.float32)
    ms = jnp.maximum(mx[0:1, :], jnp.concatenate(
        [jnp.max(a_s, axis=0, keepdims=True), z112], axis=1))
    md = jnp.maximum(mx[1:2, :], jnp.concatenate(
        [jnp.max(a_d, axis=0, keepdims=True), z112], axis=1))
    mx_ref[...] = jnp.concatenate([ms, md, mx[2:8, :]], axis=0)


_k1 = pl.pallas_call(
    _k1_body,
    grid=(NGRID,),
    in_specs=[
        pl.BlockSpec((NBK, F), lambda i: (i, 0)),
        pl.BlockSpec((F, F), lambda i: (0, 0)),
        pl.BlockSpec((F, 16), lambda i: (0, 0)),
        pl.BlockSpec((F, 16), lambda i: (0, 0)),
    ],
    out_specs=[
        pl.BlockSpec((NT, NBK, 128), lambda i: (0, i, 0)),
        pl.BlockSpec((NBK, 128), lambda i: (i, 0)),
        pl.BlockSpec((NBK, 128), lambda i: (i, 0)),
        pl.BlockSpec((8, 128), lambda i: (0, 0)),
    ],
    out_shape=[
        jax.ShapeDtypeStruct((NT, N, 128), jnp.float32),
        jax.ShapeDtypeStruct((N, 128), jnp.float32),
        jax.ShapeDtypeStruct((N, 128), jnp.float32),
        jax.ShapeDtypeStruct((8, 128), jnp.float32),
    ],
)


_mesh = plsc.VectorSubcoreMesh(core_axis_name="c", subcore_axis_name="s")
_sc_params = pltpu.CompilerParams(use_tc_tiling_on_sc=False)


@functools.partial(
    pl.kernel,
    out_type=[
        jax.ShapeDtypeStruct((EPAD, 16), jnp.float32),
        jax.ShapeDtypeStruct((2, N, 16), jnp.float32),
    ],
    mesh=_mesh,
    scratch_types=[
        pltpu.VMEM((RPT, 128), jnp.int32),
        pltpu.VMEM((RPT, 128), jnp.int32),
        pltpu.VMEM((128, 128), jnp.float32),
        pltpu.VMEM((128, 128), jnp.float32),
        pltpu.VMEM((128, 16), jnp.float32),
        pltpu.VMEM((NPT, 16), jnp.float32),
        pltpu.VMEM((8, 128), jnp.float32),
        pltpu.VMEM_SHARED((N, 16), jnp.float32),
        pltpu.SemaphoreType.DMA,
    ],
    compiler_params=_sc_params,
)
def _p1(ats_hbm, atd_hbm, mx_hbm, src_hbm, dst_hbm, ex_hbm, den_hbm,
        srct, dstt, a1, a2, exb, zbuf, mxv, den_sh, sem):
    cid = lax.axis_index("c")
    sid = lax.axis_index("s")
    wid = cid * 16 + sid

    def zrow(r, c):
        zbuf[r, :] = jnp.zeros((16,), jnp.float32)
        return c

    lax.fori_loop(0, NPT, zrow, 0)
    pltpu.sync_copy(zbuf, den_sh.at[pl.ds(sid * NPT, NPT)])
    plsc.subcore_barrier()

    pltpu.sync_copy(mx_hbm, mxv)
    pltpu.sync_copy(src_hbm.at[pl.ds(wid * RPT, RPT)], srct)
    pltpu.sync_copy(dst_hbm.at[pl.ds(wid * RPT, RPT)], dstt)
    msum = mxv[0, pl.ds(0, 16)] + mxv[1, pl.ds(0, 16)]
    mvec = jnp.maximum(msum, 0.2 * msum)
    lane_mask = jnp.where(lax.iota(jnp.int32, 16) < 8,
                          jnp.float32(1.0), jnp.float32(0.0))

    def batch(b, carry):
        ebase = wid * EPT + b * 128
        pltpu.async_copy(ats_hbm.at[srct.at[b]], a1, sem).wait()
        pltpu.async_copy(atd_hbm.at[dstt.at[b]], a2, sem).wait()

        def erow(e, c):
            av = a1[e, pl.ds(0, 16)] + a2[e, pl.ds(0, 16)]
            al = jnp.maximum(av, 0.2 * av)
            valid = jnp.where(ebase + e < ET, jnp.float32(1.0),
                              jnp.float32(0.0))
            exb[e, :] = jnp.exp(al - mvec) * lane_mask * valid
            return c

        lax.fori_loop(0, 128, erow, 0)
        pltpu.sync_copy(exb, ex_hbm.at[pl.ds(ebase, 128)])
        pltpu.sync_copy(exb, den_sh.at[dstt.at[b]], add=True)
        return carry

    lax.fori_loop(0, RPT, batch, 0)

    plsc.subcore_barrier()
    pltpu.sync_copy(den_sh.at[pl.ds(sid * NPT, NPT)], zbuf)
    pltpu.sync_copy(zbuf, den_hbm.at[cid, pl.ds(sid * NPT, NPT)])


def _vsplat(v, idx):
    dn = lax.GatherDimensionNumbers(
        offset_dims=(), collapsed_slice_dims=(0,), start_index_map=(0,))
    return lax.gather(v, idx[:, None], dn, slice_sizes=(1,),
                      mode=lax.GatherScatterMode.PROMISE_IN_BOUNDS)


@functools.partial(
    pl.kernel,
    out_type=jax.ShapeDtypeStruct((NT, N, 128), jnp.float32),
    mesh=_mesh,
    scratch_types=[
        pltpu.VMEM((1, 128), jnp.int32),
        pltpu.VMEM((1, 128), jnp.int32),
        pltpu.VMEM((1, 128), jnp.int32),
        pltpu.VMEM((128, 128), jnp.float32),
        pltpu.VMEM((128, 16), jnp.float32),
        pltpu.VMEM((NPT, 16), jnp.float32),
        pltpu.VMEM((NPT, 16), jnp.float32),
        pltpu.VMEM_SHARED((N, 128), jnp.float32),
        pltpu.SemaphoreType.DMA,
    ],
    compiler_params=_sc_params,
)
def _p2(xw_hbm, ex_hbm, den_hbm, src_hbm, dst_hbm, m_hbm,
        srcb, dstb, idx2, rows, exb, denv, dbuf, tmp_sh, sem):
    cid = lax.axis_index("c")
    sid = lax.axis_index("s")

    pltpu.sync_copy(den_hbm.at[0, pl.ds(sid * NPT, NPT)], denv)
    pltpu.sync_copy(den_hbm.at[1, pl.ds(sid * NPT, NPT)], dbuf)

    def drow(r, c):
        denv[r, :] = denv[r, :] + dbuf[r, :]
        return c

    lax.fori_loop(0, NPT, drow, 0)

    for jj in range(5):
        jt = cid * 5 + jj

        @pl.when(jt < NT)
        def _():
            def zrow(r, c):
                for l in range(8):
                    rows[r, pl.ds(l * 16, 16)] = jnp.zeros((16,),
                                                           jnp.float32)
                return c

            lax.fori_loop(0, 128, zrow, 0)
            for (off, sz) in ((0, 128), (128, 128), (256, 128), (384, 128),
                              (512, NPT - 512)):
                pltpu.sync_copy(rows.at[pl.ds(0, sz)],
                                tmp_sh.at[pl.ds(sid * NPT + off, sz)])
            plsc.subcore_barrier()

            jtN = jt * N
            hvs = []
            for li in range(8):
                cols = jt * 128 + li * 16 + lax.iota(jnp.int32, 16)
                hvs.append(lax.shift_right_logical(cols * 16132, 21))
            zidx = jnp.zeros((16,), jnp.int32)
            hAv = _vsplat(hvs[0], zidx)
            hBv = hAv + 1
            masks = [hvs[li] == hAv for li in range(8)]

            def batch(b, carry):
                ebase = sid * EPT2 + b * 128
                pltpu.sync_copy(src_hbm.at[pl.ds(sid * RPT2 + b, 1)], srcb)
                pltpu.sync_copy(dst_hbm.at[pl.ds(sid * RPT2 + b, 1)], dstb)
                for l in range(8):
                    idx2[0, pl.ds(l * 16, 16)] = (
                        srcb[0, pl.ds(l * 16, 16)] + jtN)
                pltpu.async_copy(xw_hbm.at[idx2.at[0]], rows, sem).wait()
                pltpu.sync_copy(ex_hbm.at[pl.ds(ebase, 128)], exb)

                def erow(e, c):
                    exr = exb[e, :]
                    coefA = _vsplat(exr, hAv)
                    coefB = _vsplat(exr, hBv)
                    for li in range(8):
                        coef = jnp.where(masks[li], coefA, coefB)
                        rows[e, pl.ds(li * 16, 16)] = (
                            rows[e, pl.ds(li * 16, 16)] * coef)
                    return c

                lax.fori_loop(0, 128, erow, 0)
                pltpu.sync_copy(rows, tmp_sh.at[dstb.at[0]], add=True)
                return carry

            lax.fori_loop(0, RPT2, batch, 0)
            plsc.subcore_barrier()

            for (off, sz) in ((0, 128), (128, 128), (256, 128), (384, 128),
                              (512, NPT - 512)):
                pltpu.sync_copy(tmp_sh.at[pl.ds(sid * NPT + off, sz)],
                                rows.at[pl.ds(0, sz)])

                def nrow(r, c):
                    dr = denv[off + r, :]
                    dA = _vsplat(dr, hAv)
                    dB = _vsplat(dr, hBv)
                    for li in range(8):
                        dv = jnp.where(masks[li], dA, dB)
                        rows[r, pl.ds(li * 16, 16)] = (
                            rows[r, pl.ds(li * 16, 16)] / dv)
                    return c

                lax.fori_loop(0, sz, nrow, 0)
                pltpu.sync_copy(rows.at[pl.ds(0, sz)],
                                m_hbm.at[jt, pl.ds(sid * NPT + off, sz)])
            plsc.subcore_barrier()


def _k2a_body(m_ref, bias_ref, prev_ref, gat_ref, st_ref):
    i = pl.program_id(0)
    parts = [m_ref[j] for j in range(NT)]
    g = jnp.concatenate(parts, axis=1)[:, 0:F]
    g = g + bias_ref[...] + prev_ref[...]
    gat_ref[...] = g

    @pl.when(i == 0)
    def _():
        st_ref[...] = jnp.zeros((8, F), jnp.float32)

    s = st_ref[...]
    s0 = s[0:1] + jnp.sum(g, axis=0, keepdims=True)
    s1 = s[1:2] + jnp.sum(g * g, axis=0, keepdims=True)
    st_ref[...] = jnp.concatenate([s0, s1, s[2:8]], axis=0)


_k2a = pl.pallas_call(
    _k2a_body,
    grid=(NGRID,),
    in_specs=[
        pl.BlockSpec((NT, NBK, 128), lambda i: (0, i, 0)),
        pl.BlockSpec((1, F), lambda i: (0, 0)),
        pl.BlockSpec((NBK, F), lambda i: (i, 0)),
    ],
    out_specs=[
        pl.BlockSpec((NBK, F), lambda i: (i, 0)),
        pl.BlockSpec((8, F), lambda i: (0, 0)),
    ],
    out_shape=[
        jax.ShapeDtypeStruct((N, F), jnp.float32),
        jax.ShapeDtypeStruct((8, F), jnp.float32),
    ],
)


def _k2b_body(gat_ref, st_ref, gam_ref, bet_ref, out_ref):
    s = st_ref[...]
    mu = s[0:1] / N
    var = s[1:2] / N - mu * mu
    inv = lax.rsqrt(var + 1e-5)
    out_ref[...] = jnp.maximum(
        (gat_ref[...] - mu) * inv * gam_ref[...] + bet_ref[...], 0.0)


_k2b = pl.pallas_call(
    _k2b_body,
    grid=(NGRID,),
    in_specs=[
        pl.BlockSpec((NBK, F), lambda i: (i, 0)),
        pl.BlockSpec((8, F), lambda i: (0, 0)),
        pl.BlockSpec((1, F), lambda i: (0, 0)),
        pl.BlockSpec((1, F), lambda i: (0, 0)),
    ],
    out_specs=pl.BlockSpec((NBK, F), lambda i: (i, 0)),
    out_shape=jax.ShapeDtypeStruct((N, F), jnp.float32),
)


def _k3_body(h_ref, b_ref, wo_ref, bo_ref, out_ref, acc_ref, cnt_ref):
    i = pl.program_id(0)

    @pl.when(i == 0)
    def _():
        acc_ref[...] = jnp.zeros((G, F), jnp.float32)
        cnt_ref[...] = jnp.zeros((G, 128), jnp.float32)

    ids = b_ref[0, 0, :]
    gid = lax.broadcasted_iota(jnp.int32, (G, NBK), 0)
    oh = (gid == ids[None, :]).astype(jnp.float32)
    acc_ref[...] = acc_ref[...] + jnp.dot(
        oh, h_ref[...], preferred_element_type=jnp.float32,
        precision=lax.Precision.HIGHEST)
    cnt_ref[...] = cnt_ref[...] + jnp.dot(
        oh, jnp.ones((NBK, 128), jnp.float32),
        preferred_element_type=jnp.float32,
        precision=lax.Precision.HIGHEST)

    @pl.when(i == NGRID - 1)
    def _():
        pooled = acc_ref[...] / jnp.maximum(cnt_ref[:, 0:1], 1.0)
        out_ref[...] = jnp.dot(
            pooled, wo_ref[...], preferred_element_type=jnp.float32
        ) + bo_ref[...]


_k3 = pl.pallas_call(
    _k3_body,
    grid=(NGRID,),
    in_specs=[
        pl.BlockSpec((NBK, F), lambda i: (i, 0)),
        pl.BlockSpec((1, 1, NBK), lambda i: (i, 0, 0)),
        pl.BlockSpec((F, 128), lambda i: (0, 0)),
        pl.BlockSpec((1, 128), lambda i: (0, 0)),
    ],
    out_specs=[
        pl.BlockSpec((G, 128), lambda i: (0, 0)),
        pl.BlockSpec((G, F), lambda i: (0, 0)),
        pl.BlockSpec((G, 128), lambda i: (0, 0)),
    ],
    out_shape=[
        jax.ShapeDtypeStruct((G, 128), jnp.float32),
        jax.ShapeDtypeStruct((G, F), jnp.float32),
        jax.ShapeDtypeStruct((G, 128), jnp.float32),
    ],
)


def kernel(x, edge_index, batch, W0, att_src0, att_dst0, b0, gamma0, beta0,
           W1, att_src1, att_dst1, b1, gamma1, beta1, Wout, bout):
    loop = jnp.arange(N, dtype=edge_index.dtype)
    pad = jnp.zeros((EPAD - ET,), edge_index.dtype)
    src = jnp.concatenate([edge_index[0], loop, pad]).reshape(EPAD // 128,
                                                              128)
    dst = jnp.concatenate([edge_index[1], loop, pad]).reshape(EPAD // 128,
                                                              128)

    eye = jnp.eye(H, 16, dtype=jnp.float32)

    def prep(a_s, a_d):
        As = (a_s[:, :, None] * eye[:, None, :]).reshape(F, 16)
        Ad = (a_d[:, :, None] * eye[:, None, :]).reshape(F, 16)
        return As, Ad

    params = [prep(att_src0, att_dst0) + (W0, b0, gamma0, beta0),
              prep(att_src1, att_dst1) + (W1, b1, gamma1, beta1)]

    h = x
    prev = jnp.zeros_like(x)
    for (As, Ad, W, bb, ga, be) in params:
        xwt, ats, atd, mx = _k1(h, W, As, Ad)
        ex, den2 = _p1(ats, atd, mx, src, dst)
        m = _p2(xwt.reshape(NT * N, 128), ex, den2, src, dst)
        gat, st = _k2a(m, bb.reshape(1, F), prev)
        hn = _k2b(gat, st, ga.reshape(1, F), be.reshape(1, F))
        prev = h
        h = hn

    Wop = jnp.pad(Wout, ((0, 0), (0, 126)))
    bop = jnp.pad(bout, (0, 126)).reshape(1, 128)
    out = _k3(h, batch.reshape(NGRID, 1, NBK), Wop, bop)[0]
    return out[:, 0:2]


# speedup vs baseline: 1.0186x; 1.0186x over previous
import functools

import jax
import jax.numpy as jnp
from jax import lax
from jax.experimental import pallas as pl
from jax.experimental.pallas import tpu as pltpu
from jax.experimental.pallas import tpu_sc as plsc

N = 10000
F = 1040
H = 8
C = 130
NT = 9
FPP = NT * 128
G = 64
ET = 170000
NW = 32
EPT = 6144
EPAD = EPT * NW
RPT = EPT // 128
NBK = 400
NGRID = N // NBK
NPT = N // 16
EPT2 = EPAD // 16
RPT2 = EPT2 // 128


def _k1_body(h_ref, w_ref, as_ref, ad_ref, xwt_ref, ats_ref, atd_ref,
             mx_ref):
    i = pl.program_id(0)
    xw = jnp.dot(h_ref[...], w_ref[...], preferred_element_type=jnp.float32,
        precision=lax.Precision.HIGHEST)
    a_s = jnp.dot(xw, as_ref[...], preferred_element_type=jnp.float32,
        precision=lax.Precision.HIGHEST)
    a_d = jnp.dot(xw, ad_ref[...], preferred_element_type=jnp.float32,
        precision=lax.Precision.HIGHEST)
    zpad = jnp.zeros((NBK, 112), jnp.float32)
    ats_ref[...] = jnp.concatenate([a_s, zpad], axis=1)
    atd_ref[...] = jnp.concatenate([a_d, zpad], axis=1)
    xwp = jnp.concatenate([xw, jnp.zeros((NBK, FPP - F), jnp.float32)],
                          axis=1)
    for j in range(NT):
        xwt_ref[j, :, :] = xwp[:, j * 128:(j + 1) * 128]

    @pl.when(i == 0)
    def _():
        mx_ref[...] = jnp.full((8, 128), -jnp.inf, jnp.float32)

    mx = mx_ref[...]
    z112 = jnp.zeros((1, 112), jnp.float32)
    ms = jnp.maximum(mx[0:1, :], jnp.concatenate(
        [jnp.max(a_s, axis=0, keepdims=True), z112], axis=1))
    md = jnp.maximum(mx[1:2, :], jnp.concatenate(
        [jnp.max(a_d, axis=0, keepdims=True), z112], axis=1))
    mx_ref[...] = jnp.concatenate([ms, md, mx[2:8, :]], axis=0)


_k1 = pl.pallas_call(
    _k1_body,
    grid=(NGRID,),
    in_specs=[
        pl.BlockSpec((NBK, F), lambda i: (i, 0)),
        pl.BlockSpec((F, F), lambda i: (0, 0)),
        pl.BlockSpec((F, 16), lambda i: (0, 0)),
        pl.BlockSpec((F, 16), lambda i: (0, 0)),
    ],
    out_specs=[
        pl.BlockSpec((NT, NBK, 128), lambda i: (0, i, 0)),
        pl.BlockSpec((NBK, 128), lambda i: (i, 0)),
        pl.BlockSpec((NBK, 128), lambda i: (i, 0)),
        pl.BlockSpec((8, 128), lambda i: (0, 0)),
    ],
    out_shape=[
        jax.ShapeDtypeStruct((NT, N, 128), jnp.float32),
        jax.ShapeDtypeStruct((N, 128), jnp.float32),
        jax.ShapeDtypeStruct((N, 128), jnp.float32),
        jax.ShapeDtypeStruct((8, 128), jnp.float32),
    ],
)


_mesh = plsc.VectorSubcoreMesh(core_axis_name="c", subcore_axis_name="s")
_sc_params = pltpu.CompilerParams(use_tc_tiling_on_sc=False)


@functools.partial(
    pl.kernel,
    out_type=[
        jax.ShapeDtypeStruct((EPAD, 16), jnp.float32),
        jax.ShapeDtypeStruct((2, N, 16), jnp.float32),
    ],
    mesh=_mesh,
    scratch_types=[
        pltpu.VMEM((RPT, 128), jnp.int32),
        pltpu.VMEM((RPT, 128), jnp.int32),
        pltpu.VMEM((128, 128), jnp.float32),
        pltpu.VMEM((128, 128), jnp.float32),
        pltpu.VMEM((128, 16), jnp.float32),
        pltpu.VMEM((NPT, 16), jnp.float32),
        pltpu.VMEM((8, 128), jnp.float32),
        pltpu.VMEM_SHARED((N, 16), jnp.float32),
        pltpu.SemaphoreType.DMA,
    ],
    compiler_params=_sc_params,
)
def _p1(ats_hbm, atd_hbm, mx_hbm, src_hbm, dst_hbm, ex_hbm, den_hbm,
        srct, dstt, a1, a2, exb, zbuf, mxv, den_sh, sem):
    cid = lax.axis_index("c")
    sid = lax.axis_index("s")
    wid = cid * 16 + sid

    def zrow(r, c):
        zbuf[r, :] = jnp.zeros((16,), jnp.float32)
        return c

    lax.fori_loop(0, NPT, zrow, 0)
    pltpu.sync_copy(zbuf, den_sh.at[pl.ds(sid * NPT, NPT)])
    plsc.subcore_barrier()

    pltpu.sync_copy(mx_hbm, mxv)
    pltpu.sync_copy(src_hbm.at[pl.ds(wid * RPT, RPT)], srct)
    pltpu.sync_copy(dst_hbm.at[pl.ds(wid * RPT, RPT)], dstt)
    msum = mxv[0, pl.ds(0, 16)] + mxv[1, pl.ds(0, 16)]
    mvec = jnp.maximum(msum, 0.2 * msum)
    lane_mask = jnp.where(lax.iota(jnp.int32, 16) < 8,
                          jnp.float32(1.0), jnp.float32(0.0))

    def batch(b, carry):
        ebase = wid * EPT + b * 128
        pltpu.async_copy(ats_hbm.at[srct.at[b]], a1, sem).wait()
        pltpu.async_copy(atd_hbm.at[dstt.at[b]], a2, sem).wait()

        def erow(e, c):
            av = a1[e, pl.ds(0, 16)] + a2[e, pl.ds(0, 16)]
            al = jnp.maximum(av, 0.2 * av)
            valid = jnp.where(ebase + e < ET, jnp.float32(1.0),
                              jnp.float32(0.0))
            exb[e, :] = jnp.exp(al - mvec) * lane_mask * valid
            return c

        lax.fori_loop(0, 128, erow, 0)
        pltpu.sync_copy(exb, ex_hbm.at[pl.ds(ebase, 128)])
        pltpu.sync_copy(exb, den_sh.at[dstt.at[b]], add=True)
        return carry

    lax.fori_loop(0, RPT, batch, 0)

    plsc.subcore_barrier()
    pltpu.sync_copy(den_sh.at[pl.ds(sid * NPT, NPT)], zbuf)
    pltpu.sync_copy(zbuf, den_hbm.at[cid, pl.ds(sid * NPT, NPT)])


def _vsplat(v, idx):
    dn = lax.GatherDimensionNumbers(
        offset_dims=(), collapsed_slice_dims=(0,), start_index_map=(0,))
    return lax.gather(v, idx[:, None], dn, slice_sizes=(1,),
                      mode=lax.GatherScatterMode.PROMISE_IN_BOUNDS)


@functools.partial(
    pl.kernel,
    out_type=jax.ShapeDtypeStruct((NT, N, 128), jnp.float32),
    mesh=_mesh,
    scratch_types=[
        pltpu.VMEM((2, 128), jnp.int32),
        pltpu.VMEM((2, 128), jnp.int32),
        pltpu.VMEM((2, 128), jnp.int32),
        pltpu.VMEM((256, 128), jnp.float32),
        pltpu.VMEM((256, 16), jnp.float32),
        pltpu.VMEM((128, 16), jnp.float32),
        pltpu.VMEM((128, 16), jnp.float32),
        pltpu.VMEM_SHARED((N, 128), jnp.float32),
        pltpu.SemaphoreType.DMA,
    ],
    compiler_params=_sc_params,
)
def _p2(xw_hbm, ex_hbm, den_hbm, src_hbm, dst_hbm, m_hbm,
        srcb, dstb, idx2, rows, exb, denv, dbuf, tmp_sh, sem):
    cid = lax.axis_index("c")
    sid = lax.axis_index("s")

    for jj in range(5):
        jt = cid * 5 + jj

        @pl.when(jt < NT)
        def _():
            def zrow(r, c):
                for l in range(8):
                    rows[r, pl.ds(l * 16, 16)] = jnp.zeros((16,),
                                                           jnp.float32)
                return c

            lax.fori_loop(0, 128, zrow, 0)
            for (off, sz) in ((0, 128), (128, 128), (256, 128), (384, 128),
                              (512, NPT - 512)):
                pltpu.sync_copy(rows.at[pl.ds(0, sz)],
                                tmp_sh.at[pl.ds(sid * NPT + off, sz)])
            plsc.subcore_barrier()

            jtN = jt * N
            hvs = []
            for li in range(8):
                cols = jt * 128 + li * 16 + lax.iota(jnp.int32, 16)
                hvs.append(lax.shift_right_logical(cols * 16132, 21))
            zidx = jnp.zeros((16,), jnp.int32)
            hAv = _vsplat(hvs[0], zidx)
            hBv = hAv + 1
            masks = [hvs[li] == hAv for li in range(8)]

            def batch(b, carry):
                ebase = sid * EPT2 + b * 256
                pltpu.sync_copy(
                    src_hbm.at[pl.ds(sid * RPT2 + b * 2, 2)], srcb)
                pltpu.sync_copy(
                    dst_hbm.at[pl.ds(sid * RPT2 + b * 2, 2)], dstb)
                for r in range(2):
                    for l in range(8):
                        idx2[r, pl.ds(l * 16, 16)] = (
                            srcb[r, pl.ds(l * 16, 16)] + jtN)
                h1 = pltpu.async_copy(
                    xw_hbm.at[idx2.at[0]], rows.at[pl.ds(0, 128)], sem)
                h2 = pltpu.async_copy(
                    xw_hbm.at[idx2.at[1]], rows.at[pl.ds(128, 128)], sem)
                h1.wait()
                h2.wait()
                pltpu.sync_copy(ex_hbm.at[pl.ds(ebase, 256)], exb)

                def erow(e, c):
                    exr = exb[e, :]
                    coefA = _vsplat(exr, hAv)
                    coefB = _vsplat(exr, hBv)
                    for li in range(8):
                        coef = jnp.where(masks[li], coefA, coefB)
                        rows[e, pl.ds(li * 16, 16)] = (
                            rows[e, pl.ds(li * 16, 16)] * coef)
                    return c

                lax.fori_loop(0, 256, erow, 0)
                pltpu.sync_copy(rows.at[pl.ds(0, 128)],
                                tmp_sh.at[dstb.at[0]], add=True)
                pltpu.sync_copy(rows.at[pl.ds(128, 128)],
                                tmp_sh.at[dstb.at[1]], add=True)
                return carry

            lax.fori_loop(0, RPT2 // 2, batch, 0)
            plsc.subcore_barrier()

            for (off, sz) in ((0, 128), (128, 128), (256, 128), (384, 128),
                              (512, NPT - 512)):
                pltpu.sync_copy(tmp_sh.at[pl.ds(sid * NPT + off, sz)],
                                rows.at[pl.ds(0, sz)])
                pltpu.sync_copy(
                    den_hbm.at[0, pl.ds(sid * NPT + off, sz)],
                    denv.at[pl.ds(0, sz)])
                pltpu.sync_copy(
                    den_hbm.at[1, pl.ds(sid * NPT + off, sz)],
                    dbuf.at[pl.ds(0, sz)])

                def nrow(r, c):
                    dr = denv[r, :] + dbuf[r, :]
                    dA = _vsplat(dr, hAv)
                    dB = _vsplat(dr, hBv)
                    for li in range(8):
                        dv = jnp.where(masks[li], dA, dB)
                        rows[r, pl.ds(li * 16, 16)] = (
                            rows[r, pl.ds(li * 16, 16)] / dv)
                    return c

                lax.fori_loop(0, sz, nrow, 0)
                pltpu.sync_copy(rows.at[pl.ds(0, sz)],
                                m_hbm.at[jt, pl.ds(sid * NPT + off, sz)])
            plsc.subcore_barrier()


def _k2a_body(m_ref, bias_ref, prev_ref, gat_ref, st_ref):
    i = pl.program_id(0)
    parts = [m_ref[j] for j in range(NT)]
    g = jnp.concatenate(parts, axis=1)[:, 0:F]
    g = g + bias_ref[...] + prev_ref[...]
    gat_ref[...] = g

    @pl.when(i == 0)
    def _():
        st_ref[...] = jnp.zeros((8, F), jnp.float32)

    s = st_ref[...]
    s0 = s[0:1] + jnp.sum(g, axis=0, keepdims=True)
    s1 = s[1:2] + jnp.sum(g * g, axis=0, keepdims=True)
    st_ref[...] = jnp.concatenate([s0, s1, s[2:8]], axis=0)


_k2a = pl.pallas_call(
    _k2a_body,
    grid=(NGRID,),
    in_specs=[
        pl.BlockSpec((NT, NBK, 128), lambda i: (0, i, 0)),
        pl.BlockSpec((1, F), lambda i: (0, 0)),
        pl.BlockSpec((NBK, F), lambda i: (i, 0)),
    ],
    out_specs=[
        pl.BlockSpec((NBK, F), lambda i: (i, 0)),
        pl.BlockSpec((8, F), lambda i: (0, 0)),
    ],
    out_shape=[
        jax.ShapeDtypeStruct((N, F), jnp.float32),
        jax.ShapeDtypeStruct((8, F), jnp.float32),
    ],
)


def _k2b_body(gat_ref, st_ref, gam_ref, bet_ref, out_ref):
    s = st_ref[...]
    mu = s[0:1] / N
    var = s[1:2] / N - mu * mu
    inv = lax.rsqrt(var + 1e-5)
    out_ref[...] = jnp.maximum(
        (gat_ref[...] - mu) * inv * gam_ref[...] + bet_ref[...], 0.0)


_k2b = pl.pallas_call(
    _k2b_body,
    grid=(NGRID,),
    in_specs=[
        pl.BlockSpec((NBK, F), lambda i: (i, 0)),
        pl.BlockSpec((8, F), lambda i: (0, 0)),
        pl.BlockSpec((1, F), lambda i: (0, 0)),
        pl.BlockSpec((1, F), lambda i: (0, 0)),
    ],
    out_specs=pl.BlockSpec((NBK, F), lambda i: (i, 0)),
    out_shape=jax.ShapeDtypeStruct((N, F), jnp.float32),
)


def _k3_body(h_ref, b_ref, wo_ref, bo_ref, out_ref, acc_ref, cnt_ref):
    i = pl.program_id(0)

    @pl.when(i == 0)
    def _():
        acc_ref[...] = jnp.zeros((G, F), jnp.float32)
        cnt_ref[...] = jnp.zeros((G, 128), jnp.float32)

    ids = b_ref[0, 0, :]
    gid = lax.broadcasted_iota(jnp.int32, (G, NBK), 0)
    oh = (gid == ids[None, :]).astype(jnp.float32)
    acc_ref[...] = acc_ref[...] + jnp.dot(
        oh, h_ref[...], preferred_element_type=jnp.float32,
        precision=lax.Precision.HIGHEST)
    cnt_ref[...] = cnt_ref[...] + jnp.dot(
        oh, jnp.ones((NBK, 128), jnp.float32),
        preferred_element_type=jnp.float32,
        precision=lax.Precision.HIGHEST)

    @pl.when(i == NGRID - 1)
    def _():
        pooled = acc_ref[...] / jnp.maximum(cnt_ref[:, 0:1], 1.0)
        out_ref[...] = jnp.dot(
            pooled, wo_ref[...], preferred_element_type=jnp.float32
        ) + bo_ref[...]


_k3 = pl.pallas_call(
    _k3_body,
    grid=(NGRID,),
    in_specs=[
        pl.BlockSpec((NBK, F), lambda i: (i, 0)),
        pl.BlockSpec((1, 1, NBK), lambda i: (i, 0, 0)),
        pl.BlockSpec((F, 128), lambda i: (0, 0)),
        pl.BlockSpec((1, 128), lambda i: (0, 0)),
    ],
    out_specs=[
        pl.BlockSpec((G, 128), lambda i: (0, 0)),
        pl.BlockSpec((G, F), lambda i: (0, 0)),
        pl.BlockSpec((G, 128), lambda i: (0, 0)),
    ],
    out_shape=[
        jax.ShapeDtypeStruct((G, 128), jnp.float32),
        jax.ShapeDtypeStruct((G, F), jnp.float32),
        jax.ShapeDtypeStruct((G, 128), jnp.float32),
    ],
)


def kernel(x, edge_index, batch, W0, att_src0, att_dst0, b0, gamma0, beta0,
           W1, att_src1, att_dst1, b1, gamma1, beta1, Wout, bout):
    loop = jnp.arange(N, dtype=edge_index.dtype)
    pad = jnp.zeros((EPAD - ET,), edge_index.dtype)
    src = jnp.concatenate([edge_index[0], loop, pad]).reshape(EPAD // 128,
                                                              128)
    dst = jnp.concatenate([edge_index[1], loop, pad]).reshape(EPAD // 128,
                                                              128)

    eye = jnp.eye(H, 16, dtype=jnp.float32)

    def prep(a_s, a_d):
        As = (a_s[:, :, None] * eye[:, None, :]).reshape(F, 16)
        Ad = (a_d[:, :, None] * eye[:, None, :]).reshape(F, 16)
        return As, Ad

    params = [prep(att_src0, att_dst0) + (W0, b0, gamma0, beta0),
              prep(att_src1, att_dst1) + (W1, b1, gamma1, beta1)]

    h = x
    prev = jnp.zeros_like(x)
    for (As, Ad, W, bb, ga, be) in params:
        xwt, ats, atd, mx = _k1(h, W, As, Ad)
        ex, den2 = _p1(ats, atd, mx, src, dst)
        m = _p2(xwt.reshape(NT * N, 128), ex, den2, src, dst)
        gat, st = _k2a(m, bb.reshape(1, F), prev)
        hn = _k2b(gat, st, ga.reshape(1, F), be.reshape(1, F))
        prev = h
        h = hn

    Wop = jnp.pad(Wout, ((0, 0), (0, 126)))
    bop = jnp.pad(bout, (0, 126)).reshape(1, 128)
    out = _k3(h, batch.reshape(NGRID, 1, NBK), Wop, bop)[0]
    return out[:, 0:2]
